# baseline (speedup 1.0000x reference)
import functools

import jax
import jax.numpy as jnp
from jax import lax
from jax.experimental import pallas as pl
from jax.experimental.pallas import tpu as pltpu
from jax.experimental.pallas import tpu_sc as plsc

_B = 16384
_NF = 8
_DM = 32


@functools.cache
def _make_sc_gather():
    info = plsc.get_sparse_core_info()
    nc, ns = info.num_cores, info.num_subcores
    nw = nc * ns
    bpw = _B // nw

    mesh = plsc.VectorSubcoreMesh(core_axis_name="c", subcore_axis_name="s")

    @functools.partial(
        pl.kernel,
        mesh=mesh,
        out_type=[
            jax.ShapeDtypeStruct((_B * _NF,), jnp.float32),
            jax.ShapeDtypeStruct((_B * _NF,), jnp.float32),
            jax.ShapeDtypeStruct((_B * _DM,), jnp.float32),
            jax.ShapeDtypeStruct((_B * _DM,), jnp.float32),
        ],
        scratch_types=[
            pltpu.VMEM((bpw,), jnp.int32),
            pltpu.VMEM((bpw,), jnp.int32),
            pltpu.VMEM((bpw * _NF,), jnp.float32),
            pltpu.VMEM((bpw * _NF,), jnp.float32),
            pltpu.VMEM((bpw * _DM,), jnp.float32),
            pltpu.VMEM((bpw * _DM,), jnp.float32),
            pltpu.SemaphoreType.DMA,
            pltpu.SemaphoreType.DMA,
            pltpu.SemaphoreType.DMA,
            pltpu.SemaphoreType.DMA,
        ],
    )
    def gather(users, items, Ug, Ig, Um, Im,
               ug_o, ig_o, eu_o, ei_o,
               uv, iv, bug, big, bum, bim, s0, s1, s2, s3):
        wid = lax.axis_index("s") * nc + lax.axis_index("c")
        base = wid * bpw
        pltpu.sync_copy(users.at[pl.ds(base, bpw)], uv)
        pltpu.sync_copy(items.at[pl.ds(base, bpw)], iv)

        def body(g, _):
            uvec = uv[pl.ds(g * 16, 16)]
            ivec = iv[pl.ds(g * 16, 16)]
            for l in range(16):
                u = lax.squeeze(lax.slice(uvec, (l,), (l + 1,)), (0,))
                it = lax.squeeze(lax.slice(ivec, (l,), (l + 1,)), (0,))
                j = g * 16 + l
                pltpu.make_async_copy(
                    Ug.at[pl.ds(u * _NF, _NF)],
                    bug.at[pl.ds(j * _NF, _NF)], s0).start()
                pltpu.make_async_copy(
                    Um.at[pl.ds(u * _DM, _DM)],
                    bum.at[pl.ds(j * _DM, _DM)], s2).start()
                pltpu.make_async_copy(
                    Ig.at[pl.ds(it * _NF, _NF)],
                    big.at[pl.ds(j * _NF, _NF)], s1).start()
                pltpu.make_async_copy(
                    Im.at[pl.ds(it * _DM, _DM)],
                    bim.at[pl.ds(j * _DM, _DM)], s3).start()
            return ()

        lax.fori_loop(0, bpw // 16, body, ())
        pltpu.make_async_copy(Ug.at[pl.ds(0, bpw * _NF)], bug, s0).wait()
        pltpu.make_async_copy(Ig.at[pl.ds(0, bpw * _NF)], big, s1).wait()
        pltpu.make_async_copy(Um.at[pl.ds(0, bpw * _DM)], bum, s2).wait()
        pltpu.make_async_copy(Im.at[pl.ds(0, bpw * _DM)], bim, s3).wait()
        pltpu.sync_copy(bug, ug_o.at[pl.ds(base * _NF, bpw * _NF)])
        pltpu.sync_copy(big, ig_o.at[pl.ds(base * _NF, bpw * _NF)])
        pltpu.sync_copy(bum, eu_o.at[pl.ds(base * _DM, bpw * _DM)])
        pltpu.sync_copy(bim, ei_o.at[pl.ds(base * _DM, bpw * _DM)])

    return gather


_BLK = 2048


def _mlp_body(ug_r, ig_r, eu_r, ei_r, w1_r, b1_r, w2_r, b2_r, w3_r, b3_r,
              wl_r, bl_r, o_r):
    dn = (((1,), (1,)), ((), ()))
    f32 = jnp.float32
    g = ug_r[...] * ig_r[...]
    w1 = w1_r[...]
    h = lax.dot_general(eu_r[...], w1[:, :_DM], dn, preferred_element_type=f32)
    h = h + lax.dot_general(ei_r[...], w1[:, _DM:], dn, preferred_element_type=f32)
    h = jnp.maximum(h + b1_r[...], 0.0)
    h = lax.dot_general(h, w2_r[...], dn, preferred_element_type=f32)
    h = jnp.maximum(h + b2_r[...], 0.0)
    h = lax.dot_general(h, w3_r[...], dn, preferred_element_type=f32)
    h = jnp.maximum(h + b3_r[...], 0.0)
    wl = wl_r[...]
    out = lax.dot_general(g, wl[:, :_NF], dn, preferred_element_type=f32)
    out = out + lax.dot_general(h, wl[:, _NF:], dn, preferred_element_type=f32)
    o_r[...] = out + bl_r[...]


def _mlp(ug, ig, eu, ei, W1, b1, W2, b2, W3, b3, Wl, bl):
    def full(shape):
        nd = len(shape)
        return pl.BlockSpec(shape, lambda i: (0,) * nd)

    grid = _B // _BLK
    return pl.pallas_call(
        _mlp_body,
        grid=(grid,),
        in_specs=[
            pl.BlockSpec((_BLK, _NF), lambda i: (i, 0)),
            pl.BlockSpec((_BLK, _NF), lambda i: (i, 0)),
            pl.BlockSpec((_BLK, _DM), lambda i: (i, 0)),
            pl.BlockSpec((_BLK, _DM), lambda i: (i, 0)),
            full(W1.shape), full((1, 32)), full(W2.shape), full((1, 16)),
            full(W3.shape), full((1, 8)), full(Wl.shape), full((1, 1)),
        ],
        out_specs=pl.BlockSpec((_BLK, 1), lambda i: (i, 0)),
        out_shape=jax.ShapeDtypeStruct((_B, 1), jnp.float32),
    )(ug, ig, eu, ei, W1, b1.reshape(1, -1), W2, b2.reshape(1, -1),
      W3, b3.reshape(1, -1), Wl, bl.reshape(1, -1))


def kernel(users, items, Ug, Ig, Um, Im, W1, b1, W2, b2, W3, b3, Wl, bl):
    ug1, ig1, eu1, ei1 = _make_sc_gather()(
        users, items, Ug.reshape(-1), Ig.reshape(-1),
        Um.reshape(-1), Im.reshape(-1))
    out = _mlp(ug1.reshape(_B, _NF), ig1.reshape(_B, _NF),
               eu1.reshape(_B, _DM), ei1.reshape(_B, _DM),
               W1, b1, W2, b2, W3, b3, Wl, bl)
    return out.reshape(-1)


# speedup vs baseline: 1.4729x; 1.4729x over previous
import functools

import jax
import jax.numpy as jnp
from jax import lax
from jax.experimental import pallas as pl
from jax.experimental.pallas import tpu as pltpu
from jax.experimental.pallas import tpu_sc as plsc

_B = 16384
_NF = 8
_DM = 32


@functools.cache
def _make_sc_gather():
    info = plsc.get_sparse_core_info()
    nc, ns = info.num_cores, info.num_subcores
    nw = nc * ns
    bpw = _B // nw
    mesh = plsc.VectorSubcoreMesh(core_axis_name="c", subcore_axis_name="s")

    @functools.partial(
        pl.kernel,
        mesh=mesh,
        compiler_params=pltpu.CompilerParams(needs_layout_passes=False),
        out_type=[
            jax.ShapeDtypeStruct((_B, _NF), jnp.float32),
            jax.ShapeDtypeStruct((_B, _NF), jnp.float32),
            jax.ShapeDtypeStruct((_B, _DM), jnp.float32),
            jax.ShapeDtypeStruct((_B, _DM), jnp.float32),
        ],
        scratch_types=[
            pltpu.VMEM((bpw,), jnp.int32),
            pltpu.VMEM((bpw,), jnp.int32),
            pltpu.VMEM((128, _NF), jnp.float32),
            pltpu.VMEM((128, _NF), jnp.float32),
            pltpu.VMEM((128, _DM), jnp.float32),
            pltpu.VMEM((128, _DM), jnp.float32),
            pltpu.SemaphoreType.DMA,
            pltpu.SemaphoreType.DMA,
            pltpu.SemaphoreType.DMA,
            pltpu.SemaphoreType.DMA,
        ],
    )
    def gather(users, items, Ug, Ig, Um, Im,
               ug_o, ig_o, eu_o, ei_o,
               uv, iv, bug, big, bum, bim, s0, s1, s2, s3):
        wid = lax.axis_index("s") * nc + lax.axis_index("c")
        base = wid * bpw
        pltpu.sync_copy(users.at[pl.ds(base, bpw)], uv)
        pltpu.sync_copy(items.at[pl.ds(base, bpw)], iv)
        lanes = lax.iota(jnp.int32, 16)

        for c in range(bpw // 128):
            def body(g, _):
                uvec = uv[pl.ds(c * 128 + g * 16, 16)]
                ivec = iv[pl.ds(c * 128 + g * 16, 16)]
                for l in range(16):
                    u = jnp.sum(jnp.where(lanes == l, uvec, 0))
                    it = jnp.sum(jnp.where(lanes == l, ivec, 0))
                    j = g * 16 + l
                    pltpu.make_async_copy(
                        Ug.at[pl.ds(u, 1)], bug.at[pl.ds(j, 1)], s0).start()
                    pltpu.make_async_copy(
                        Um.at[pl.ds(u, 1)], bum.at[pl.ds(j, 1)], s2).start()
                    pltpu.make_async_copy(
                        Ig.at[pl.ds(it, 1)], big.at[pl.ds(j, 1)], s1).start()
                    pltpu.make_async_copy(
                        Im.at[pl.ds(it, 1)], bim.at[pl.ds(j, 1)], s3).start()
                return ()

            lax.fori_loop(0, 8, body, ())
            pltpu.make_async_copy(Ug.at[pl.ds(0, 128)], bug, s0).wait()
            pltpu.make_async_copy(Ig.at[pl.ds(0, 128)], big, s1).wait()
            pltpu.make_async_copy(Um.at[pl.ds(0, 128)], bum, s2).wait()
            pltpu.make_async_copy(Im.at[pl.ds(0, 128)], bim, s3).wait()
            osl = pl.ds(base + c * 128, 128)
            pltpu.sync_copy(bug, ug_o.at[osl])
            pltpu.sync_copy(big, ig_o.at[osl])
            pltpu.sync_copy(bum, eu_o.at[osl])
            pltpu.sync_copy(bim, ei_o.at[osl])

    return gather


_BLK = 2048


def _mlp_body(ug_r, ig_r, eu_r, ei_r, w1_r, b1_r, w2_r, b2_r, w3_r, b3_r,
              wl_r, bl_r, o_r):
    dn = (((1,), (1,)), ((), ()))
    f32 = jnp.float32
    g = ug_r[...] * ig_r[...]
    w1 = w1_r[...]
    h = lax.dot_general(eu_r[...], w1[:, :_DM], dn, preferred_element_type=f32)
    h = h + lax.dot_general(ei_r[...], w1[:, _DM:], dn, preferred_element_type=f32)
    h = jnp.maximum(h + b1_r[...], 0.0)
    h = lax.dot_general(h, w2_r[...], dn, preferred_element_type=f32)
    h = jnp.maximum(h + b2_r[...], 0.0)
    h = lax.dot_general(h, w3_r[...], dn, preferred_element_type=f32)
    h = jnp.maximum(h + b3_r[...], 0.0)
    wl = wl_r[...]
    out = lax.dot_general(g, wl[:, :_NF], dn, preferred_element_type=f32)
    out = out + lax.dot_general(h, wl[:, _NF:], dn, preferred_element_type=f32)
    o_r[...] = out + bl_r[...]


def _mlp(ug, ig, eu, ei, W1, b1, W2, b2, W3, b3, Wl, bl):
    def full(shape):
        nd = len(shape)
        return pl.BlockSpec(shape, lambda i: (0,) * nd)

    grid = _B // _BLK
    return pl.pallas_call(
        _mlp_body,
        grid=(grid,),
        in_specs=[
            pl.BlockSpec((_BLK, _NF), lambda i: (i, 0)),
            pl.BlockSpec((_BLK, _NF), lambda i: (i, 0)),
            pl.BlockSpec((_BLK, _DM), lambda i: (i, 0)),
            pl.BlockSpec((_BLK, _DM), lambda i: (i, 0)),
            full(W1.shape), full((1, 32)), full(W2.shape), full((1, 16)),
            full(W3.shape), full((1, 8)), full(Wl.shape), full((1, 1)),
        ],
        out_specs=pl.BlockSpec((_BLK, 1), lambda i: (i, 0)),
        out_shape=jax.ShapeDtypeStruct((_B, 1), jnp.float32),
    )(ug, ig, eu, ei, W1, b1.reshape(1, -1), W2, b2.reshape(1, -1),
      W3, b3.reshape(1, -1), Wl, bl.reshape(1, -1))


def kernel(users, items, Ug, Ig, Um, Im, W1, b1, W2, b2, W3, b3, Wl, bl):
    ug, ig, eu, ei = _make_sc_gather()(users, items, Ug, Ig, Um, Im)
    out = _mlp(ug, ig, eu, ei, W1, b1, W2, b2, W3, b3, Wl, bl)
    return out.reshape(-1)
